# baseline (speedup 1.0000x reference)
import jax
import jax.numpy as jnp
from jax.experimental import pallas as pl
from jax.experimental.pallas import tpu as pltpu

_B, _N, _M, _D = 4, 4096, 4096, 3
_TN = 2048
_NB = _N // _TN


def _chamfer_block_kernel(u_ref, xn_ref, yt_ref, yn_ref, o1_ref, o2_ref, m2_ref):
    i = pl.program_id(1)

    u = u_ref[0]
    xn = xn_ref[0]
    yt = yt_ref[0]
    yn = yn_ref[0]

    g = None
    for k in range(_D):
        uk = u[:, k][:, None]
        ykt = yt[k, :][None, :]
        p = uk * ykt
        g = p if g is None else g + p

    t1 = g + yn
    r1 = jnp.min(t1, axis=1)[:, None] + xn
    s1 = jnp.sum(r1, keepdims=True)[:1, :1]

    t2 = g + xn
    m2 = jnp.min(t2, axis=0, keepdims=True)

    @pl.when(i == 0)
    def _init():
        o1_ref[...] = s1[None]
        m2_ref[...] = m2

    @pl.when(i > 0)
    def _acc():
        o1_ref[...] += s1[None]
        m2_ref[...] = jnp.minimum(m2_ref[...], m2)

    @pl.when(i == _NB - 1)
    def _flush_m2():
        o2_ref[...] = jnp.sum(m2_ref[...] + yn, keepdims=True)[None]


@jax.jit
def kernel(input1, input2):
    u = -2.0 * input1
    xn = jnp.sum(input1 * input1, axis=2, keepdims=True)
    yt = jnp.transpose(input2, (0, 2, 1))
    yn = jnp.sum(input2 * input2, axis=2)[:, None, :]
    s1, s2 = pl.pallas_call(
        _chamfer_block_kernel,
        grid=(_B, _NB),
        in_specs=[
            pl.BlockSpec((1, _TN, _D), lambda b, i: (b, i, 0)),
            pl.BlockSpec((1, _TN, 1), lambda b, i: (b, i, 0)),
            pl.BlockSpec((1, _D, _M), lambda b, i: (b, 0, 0)),
            pl.BlockSpec((1, 1, _M), lambda b, i: (b, 0, 0)),
        ],
        out_specs=[
            pl.BlockSpec((1, 1, 1), lambda b, i: (b, 0, 0)),
            pl.BlockSpec((1, 1, 1), lambda b, i: (b, 0, 0)),
        ],
        out_shape=[
            jax.ShapeDtypeStruct((_B, 1, 1), jnp.float32),
            jax.ShapeDtypeStruct((_B, 1, 1), jnp.float32),
        ],
        scratch_shapes=[pltpu.VMEM((1, _M), jnp.float32)],
        compiler_params=pltpu.CompilerParams(
            dimension_semantics=("parallel", "arbitrary"),
        ),
    )(u, xn, yt, yn)
    return jnp.sum(s1) * (1.0 / (_B * _N)) + jnp.sum(s2) * (1.0 / (_B * _M))


# speedup vs baseline: 1.8434x; 1.8434x over previous
import jax
import jax.numpy as jnp
from jax.experimental import pallas as pl
from jax.experimental.pallas import tpu as pltpu

_B, _N, _M, _D = 4, 4096, 4096, 3
_TN = 2048
_NB = _N // _TN


def _chamfer_block_kernel(u_ref, xn_ref, yt_ref, yn_ref, o1_ref, o2_ref, m2_ref):
    i = pl.program_id(1)

    u = u_ref[0]
    xn = xn_ref[0]
    yt = yt_ref[0]
    yn = yn_ref[0]

    g = jax.lax.dot_general(
        u, yt, (((1,), (0,)), ((), ())), preferred_element_type=jnp.float32
    )

    d = (g + xn) + yn

    r1 = jnp.min(d, axis=1)[:, None]
    s1 = jnp.sum(r1, keepdims=True)[:1, :1]

    m2 = jnp.min(d, axis=0, keepdims=True)

    @pl.when(i == 0)
    def _init():
        o1_ref[...] = s1[None]
        m2_ref[...] = m2

    @pl.when(i > 0)
    def _acc():
        o1_ref[...] += s1[None]
        m2_ref[...] = jnp.minimum(m2_ref[...], m2)

    @pl.when(i == _NB - 1)
    def _flush_m2():
        o2_ref[...] = jnp.sum(m2_ref[...], keepdims=True)[None]


@jax.jit
def kernel(input1, input2):
    u = -2.0 * input1
    xn = jnp.sum(input1 * input1, axis=2, keepdims=True)
    yt = jnp.transpose(input2, (0, 2, 1))
    yn = jnp.sum(input2 * input2, axis=2)[:, None, :]
    s1, s2 = pl.pallas_call(
        _chamfer_block_kernel,
        grid=(_B, _NB),
        in_specs=[
            pl.BlockSpec((1, _TN, _D), lambda b, i: (b, i, 0)),
            pl.BlockSpec((1, _TN, 1), lambda b, i: (b, i, 0)),
            pl.BlockSpec((1, _D, _M), lambda b, i: (b, 0, 0)),
            pl.BlockSpec((1, 1, _M), lambda b, i: (b, 0, 0)),
        ],
        out_specs=[
            pl.BlockSpec((1, 1, 1), lambda b, i: (b, 0, 0)),
            pl.BlockSpec((1, 1, 1), lambda b, i: (b, 0, 0)),
        ],
        out_shape=[
            jax.ShapeDtypeStruct((_B, 1, 1), jnp.float32),
            jax.ShapeDtypeStruct((_B, 1, 1), jnp.float32),
        ],
        scratch_shapes=[pltpu.VMEM((1, _M), jnp.float32)],
    )(u, xn, yt, yn)
    return jnp.sum(s1) * (1.0 / (_B * _N)) + jnp.sum(s2) * (1.0 / (_B * _M))
